# Initial kernel scaffold; baseline (speedup 1.0000x reference)
import jax
import jax.numpy as jnp
from jax.experimental import pallas as pl


def kernel(features, matrix, x, W, b):
    raise NotImplementedError("write your pallas kernel here")


# speedup vs baseline: 20.1441x; 20.1441x over previous
import functools

import jax
import jax.numpy as jnp
import numpy as np
from jax import lax
from jax.experimental import pallas as pl
from jax.experimental.pallas import tpu as pltpu
from jax.experimental.pallas import tpu_sc as plsc

N = 10000
D = 128
E = 320000
BATCH = 1024
FIELDS = 26

NC, NS = 2, 16
NP = 10240
EP = 327680
ER = EP // 128
ROWS_T = NP // NS

CHKA = EP // (NC * NS * 128)
CHKC = EP // (NC * NS * 128)
CHKH = CHKC // 2
XF = BATCH * FIELDS
XRP = 224
XFP = XRP * 128
CHKD = XRP // (NC * NS)

_mesh = plsc.VectorSubcoreMesh(core_axis_name="c", subcore_axis_name="s")


def _deg_body(dst_hbm, out_hbm, idx_v, ones_v, zeros_v, accum_sh):
    c = lax.axis_index("c")
    s = lax.axis_index("s")
    w = c * NS + s

    def fz(i, _):
        zeros_v[pl.ds(i * 16, 16)] = jnp.zeros((16,), jnp.float32)
        return 0

    lax.fori_loop(0, ROWS_T // 16, fz, 0)

    def fo(i, _):
        ones_v[pl.ds(i * 16, 16)] = jnp.ones((16,), jnp.float32)
        return 0

    lax.fori_loop(0, 128 // 16, fo, 0)

    pltpu.sync_copy(zeros_v, accum_sh.at[pl.ds(s * ROWS_T, ROWS_T)])
    pltpu.sync_copy(dst_hbm.at[pl.ds(w * CHKA, CHKA)], idx_v)
    plsc.subcore_barrier()

    def body(j, _):
        pltpu.sync_copy(ones_v, accum_sh.at[idx_v.at[j]], add=True)
        return 0

    lax.fori_loop(0, CHKA, body, 0)
    plsc.subcore_barrier()
    pltpu.sync_copy(accum_sh.at[pl.ds(s * ROWS_T, ROWS_T)],
                    out_hbm.at[pl.ds(c * NP + s * ROWS_T, ROWS_T)])


_deg = functools.partial(
    pl.kernel,
    out_type=jax.ShapeDtypeStruct((NC * NP,), jnp.float32),
    mesh=_mesh,
    scratch_types=[
        pltpu.VMEM((CHKA, 128), jnp.int32),
        pltpu.VMEM((128,), jnp.float32),
        pltpu.VMEM((ROWS_T,), jnp.float32),
        pltpu.VMEM_SHARED((NP,), jnp.float32),
    ],
)(_deg_body)


RB = 1024
GB = NP // RB


def _tc_body(f_ref, w_ref, dh_ref, h2z_ref, dis_ref):
    i = pl.program_id(0)
    deg = 1.0 + dh_ref[0] + dh_ref[1]
    dis = lax.rsqrt(deg)
    dis_ref[...] = dis

    @pl.when(i < GB)
    def _():
        h = jnp.dot(f_ref[...], w_ref[...],
                    preferred_element_type=jnp.float32)
        h2z_ref[...] = h * dis

    @pl.when(i >= GB)
    def _():
        h2z_ref[...] = jnp.zeros((RB, D), jnp.float32)


def _tc(feat_p, W, dh3):
    return pl.pallas_call(
        _tc_body,
        grid=(2 * GB,),
        in_specs=[
            pl.BlockSpec((RB, D), lambda i: (i % GB, 0)),
            pl.BlockSpec((D, D), lambda i: (0, 0)),
            pl.BlockSpec((2, RB, 1), lambda i: (0, i % GB, 0)),
        ],
        out_specs=[
            pl.BlockSpec((RB, D), lambda i: (i, 0)),
            pl.BlockSpec((RB, 1), lambda i: (i % GB, 0)),
        ],
        out_shape=[
            jax.ShapeDtypeStruct((NC * NP, D), jnp.float32),
            jax.ShapeDtypeStruct((NP, 1), jnp.float32),
        ],
    )(feat_p, W, dh3)


def _agg_body(src_hbm, dst_hbm, h2z_hbm, dis_hbm, b_hbm,
              p_hbm,
              sidx, didx, rowbuf, dis_v, b_v, accum_sh, sem):
    c = lax.axis_index("c")
    s = lax.axis_index("s")
    w = c * NS + s
    r0 = s * ROWS_T

    pltpu.sync_copy(h2z_hbm.at[pl.ds(c * NP + r0, ROWS_T)],
                    accum_sh.at[pl.ds(r0, ROWS_T)])
    plsc.subcore_barrier()

    def half(hf, _):
        pltpu.sync_copy(src_hbm.at[pl.ds(w * CHKC + hf * CHKH, CHKH)], sidx)
        pltpu.sync_copy(dst_hbm.at[pl.ds(w * CHKC + hf * CHKH, CHKH)], didx)

        def body(j, _):
            pltpu.async_copy(h2z_hbm.at[sidx.at[j]], rowbuf, sem).wait()
            pltpu.sync_copy(rowbuf, accum_sh.at[didx.at[j]], add=True)
            return 0

        lax.fori_loop(0, CHKH, body, 0)
        return 0

    lax.fori_loop(0, 2, half, 0)
    plsc.subcore_barrier()

    pltpu.sync_copy(b_hbm, b_v)
    bmask = jnp.where(c == 0, 1.0, 0.0).astype(jnp.float32)
    bk = [b_v[pl.ds(k * 16, 16)] * bmask for k in range(D // 16)]

    def fin_chunk(t, _):
        rbase = r0 + t * 128
        pltpu.sync_copy(accum_sh.at[pl.ds(rbase, 128)], rowbuf)
        pltpu.sync_copy(dis_hbm.at[pl.ds(rbase, 128)], dis_v)

        def fin(g, _):
            dvec = dis_v[pl.ds(g * 16, 16)]
            for rl in range(16):
                r = g * 16 + rl
                dr = dvec[rl]
                for k in range(D // 16):
                    rowbuf[r, pl.ds(k * 16, 16)] = (
                        rowbuf[r, pl.ds(k * 16, 16)] * dr + bk[k])
            return 0

        lax.fori_loop(0, 8, fin, 0)
        pltpu.sync_copy(rowbuf, p_hbm.at[pl.ds(c * NP + rbase, 128)])
        return 0

    lax.fori_loop(0, ROWS_T // 128, fin_chunk, 0)


_agg = functools.partial(
    pl.kernel,
    out_type=jax.ShapeDtypeStruct((NC * NP, D), jnp.float32),
    mesh=_mesh,
    scratch_types=[
        pltpu.VMEM((CHKH, 128), jnp.int32),
        pltpu.VMEM((CHKH, 128), jnp.int32),
        pltpu.VMEM((128, D), jnp.float32),
        pltpu.VMEM((128,), jnp.float32),
        pltpu.VMEM((D,), jnp.float32),
        pltpu.VMEM_SHARED((NP, D), jnp.float32),
        pltpu.SemaphoreType.DMA,
    ],
)(_agg_body)


def _gat_body(p_hbm, xlo_hbm, xhi_hbm, out_hbm,
              ilo_v, ihi_v, buf0, buf1, sem0, sem1):
    c = lax.axis_index("c")
    s = lax.axis_index("s")
    w = c * NS + s
    pltpu.sync_copy(xlo_hbm, ilo_v)
    pltpu.sync_copy(xhi_hbm, ihi_v)

    def body(j, _):
        cp0 = pltpu.async_copy(p_hbm.at[ilo_v.at[w * CHKD + j]], buf0, sem0)
        cp1 = pltpu.async_copy(p_hbm.at[ihi_v.at[w * CHKD + j]], buf1, sem1)
        cp0.wait()
        cp1.wait()

        def add_row(r, _):
            for k in range(D // 16):
                buf0[r, pl.ds(k * 16, 16)] = (
                    buf0[r, pl.ds(k * 16, 16)] + buf1[r, pl.ds(k * 16, 16)])
            return 0

        lax.fori_loop(0, 128, add_row, 0)
        base = (w * CHKD + j) * 128
        pltpu.sync_copy(buf0, out_hbm.at[pl.ds(base, 128)])
        return 0

    lax.fori_loop(0, CHKD, body, 0)


_gat = functools.partial(
    pl.kernel,
    out_type=jax.ShapeDtypeStruct((XFP, D), jnp.float32),
    mesh=_mesh,
    scratch_types=[
        pltpu.VMEM((XRP, 128), jnp.int32),
        pltpu.VMEM((XRP, 128), jnp.int32),
        pltpu.VMEM((128, D), jnp.float32),
        pltpu.VMEM((128, D), jnp.float32),
        pltpu.SemaphoreType.DMA,
        pltpu.SemaphoreType.DMA,
    ],
)(_gat_body)


_PAD_EDGE = (10000 + (np.arange(EP - E) % (NP - N))).astype(np.int32)
_PAD_X = (np.arange(XFP - XF) % N).astype(np.int32)


def kernel(features, matrix, x, W, b):
    feat_p = jnp.zeros((NP, D), jnp.float32).at[:N].set(features)
    pad = jnp.asarray(_PAD_EDGE)
    src = jnp.concatenate([matrix[0], pad]).reshape(ER, 128)
    dst = jnp.concatenate([matrix[1], pad]).reshape(ER, 128)

    dh = _deg(dst)
    h2z, dis2 = _tc(feat_p, W, dh.reshape(NC, NP, 1))
    dis = dis2.reshape(NP)
    p = _agg(src, dst, h2z, dis, b)
    xlo = jnp.concatenate([x.reshape(XF), jnp.asarray(_PAD_X)]).reshape(XRP, 128)
    xhi = xlo + NP
    out = _gat(p, xlo, xhi)
    return out[:XF].reshape(BATCH, FIELDS, D)


# speedup vs baseline: 25.0452x; 1.2433x over previous
import functools

import jax
import jax.numpy as jnp
import numpy as np
from jax import lax
from jax.experimental import pallas as pl
from jax.experimental.pallas import tpu as pltpu
from jax.experimental.pallas import tpu_sc as plsc

N = 10000
D = 128
E = 320000
BATCH = 1024
FIELDS = 26

NC, NS = 2, 16
NP = 10240
EP = 327680
ER = EP // 128
ROWS_T = NP // NS

NCHK = EP // (NC * NS * 128)
HCHK = NCHK // 2
XF = BATCH * FIELDS
XR = XF // 128
CHKD = XR // (NC * NS)
XTR = XR - NC * NS * CHKD

_mesh = plsc.VectorSubcoreMesh(core_axis_name="c", subcore_axis_name="s")


def _deg_body(dst_hbm, out_hbm, idx_v, ones_v, zeros_v, accum_sh):
    c = lax.axis_index("c")
    s = lax.axis_index("s")
    w = c * NS + s

    def fz(i, _):
        zeros_v[pl.ds(i * 16, 16)] = jnp.zeros((16,), jnp.float32)
        return 0

    lax.fori_loop(0, ROWS_T // 16, fz, 0)

    def fo(i, _):
        ones_v[pl.ds(i * 16, 16)] = jnp.ones((16,), jnp.float32)
        return 0

    lax.fori_loop(0, 128 // 16, fo, 0)

    pltpu.sync_copy(zeros_v, accum_sh.at[pl.ds(s * ROWS_T, ROWS_T)])
    pltpu.sync_copy(dst_hbm.at[pl.ds(w * NCHK, NCHK)], idx_v)
    plsc.subcore_barrier()

    def body(j, _):
        pltpu.sync_copy(ones_v, accum_sh.at[idx_v.at[j]], add=True)
        return 0

    lax.fori_loop(0, NCHK, body, 0)
    plsc.subcore_barrier()
    pltpu.sync_copy(accum_sh.at[pl.ds(s * ROWS_T, ROWS_T)],
                    out_hbm.at[pl.ds(c * NP + s * ROWS_T, ROWS_T)])


_deg = functools.partial(
    pl.kernel,
    out_type=jax.ShapeDtypeStruct((NC * NP,), jnp.float32),
    mesh=_mesh,
    scratch_types=[
        pltpu.VMEM((NCHK, 128), jnp.int32),
        pltpu.VMEM((128,), jnp.float32),
        pltpu.VMEM((ROWS_T,), jnp.float32),
        pltpu.VMEM_SHARED((NP,), jnp.float32),
    ],
)(_deg_body)


RB = 1024
GB = NP // RB


def _tc_body(f_ref, w_ref, dh_ref, h2z_ref, dis_ref):
    i = pl.program_id(0)
    deg = 1.0 + dh_ref[0] + dh_ref[1]
    dis = lax.rsqrt(deg)
    dis_ref[...] = dis

    @pl.when(i < GB)
    def _():
        h = jnp.dot(f_ref[...], w_ref[...],
                    preferred_element_type=jnp.float32)
        h2z_ref[...] = h * dis

    @pl.when(i >= GB)
    def _():
        h2z_ref[...] = jnp.zeros((RB, D), jnp.float32)


def _tc(feat_p, W, dh3):
    return pl.pallas_call(
        _tc_body,
        grid=(2 * GB,),
        in_specs=[
            pl.BlockSpec((RB, D), lambda i: (i % GB, 0)),
            pl.BlockSpec((D, D), lambda i: (0, 0)),
            pl.BlockSpec((2, RB, 1), lambda i: (0, i % GB, 0)),
        ],
        out_specs=[
            pl.BlockSpec((RB, D), lambda i: (i, 0)),
            pl.BlockSpec((RB, 1), lambda i: (i % GB, 0)),
        ],
        out_shape=[
            jax.ShapeDtypeStruct((NC * NP, D), jnp.float32),
            jax.ShapeDtypeStruct((NP, 1), jnp.float32),
        ],
    )(feat_p, W, dh3)


def _agg_body(src_hbm, dst_hbm, h2z_hbm, dis_hbm, b_hbm,
              p_hbm,
              sidx, didx, buf_a, buf_b, dis_v, b_v, accum_sh,
              sem_a, sem_b):
    c = lax.axis_index("c")
    s = lax.axis_index("s")
    w = c * NS + s
    r0 = s * ROWS_T

    pltpu.sync_copy(h2z_hbm.at[pl.ds(c * NP + r0, ROWS_T)],
                    accum_sh.at[pl.ds(r0, ROWS_T)])
    plsc.subcore_barrier()

    GRP = 8

    def half(hf, _):
        base = w * NCHK + hf * HCHK
        pltpu.sync_copy(src_hbm.at[pl.ds(base, HCHK)], sidx)
        pltpu.sync_copy(dst_hbm.at[pl.ds(base, HCHK)], didx)

        def group(g, _):
            gb = g * GRP
            bufs = (buf_a, buf_b)
            sems = (sem_a, sem_b)
            cp_cur = pltpu.async_copy(h2z_hbm.at[sidx.at[gb]],
                                      bufs[0], sems[0])
            for j in range(GRP):
                if j < GRP - 1:
                    nb = (j + 1) % 2
                    cp_next = pltpu.async_copy(
                        h2z_hbm.at[sidx.at[gb + j + 1]], bufs[nb], sems[nb])
                cp_cur.wait()
                pltpu.sync_copy(bufs[j % 2],
                                accum_sh.at[didx.at[gb + j]], add=True)
                if j < GRP - 1:
                    cp_cur = cp_next
            return 0

        lax.fori_loop(0, HCHK // GRP, group, 0)
        return 0

    lax.fori_loop(0, 2, half, 0)
    plsc.subcore_barrier()

    pltpu.sync_copy(b_hbm, b_v)
    bmask = jnp.where(c == 0, 1.0, 0.0).astype(jnp.float32)
    bk = [b_v[pl.ds(k * 16, 16)] * bmask for k in range(D // 16)]

    def fin_chunk(t, _):
        rbase = r0 + t * 128
        pltpu.sync_copy(accum_sh.at[pl.ds(rbase, 128)], buf_a)
        pltpu.sync_copy(dis_hbm.at[pl.ds(rbase, 128)], dis_v)

        def fin(g, _):
            dvec = dis_v[pl.ds(g * 16, 16)]
            for rl in range(16):
                r = g * 16 + rl
                dr = dvec[rl]
                for k in range(D // 16):
                    buf_a[r, pl.ds(k * 16, 16)] = (
                        buf_a[r, pl.ds(k * 16, 16)] * dr + bk[k])
            return 0

        lax.fori_loop(0, 128 // 16, fin, 0)
        pltpu.sync_copy(buf_a, p_hbm.at[pl.ds(c * NP + rbase, 128)])
        return 0

    lax.fori_loop(0, ROWS_T // 128, fin_chunk, 0)


_agg = functools.partial(
    pl.kernel,
    out_type=jax.ShapeDtypeStruct((NC * NP, D), jnp.float32),
    mesh=_mesh,
    scratch_types=[
        pltpu.VMEM((HCHK, 128), jnp.int32),
        pltpu.VMEM((HCHK, 128), jnp.int32),
        pltpu.VMEM((128, D), jnp.float32),
        pltpu.VMEM((128, D), jnp.float32),
        pltpu.VMEM((128,), jnp.float32),
        pltpu.VMEM((D,), jnp.float32),
        pltpu.VMEM_SHARED((NP, D), jnp.float32),
        pltpu.SemaphoreType.DMA,
        pltpu.SemaphoreType.DMA,
    ],
)(_agg_body)


def _gat_body(p_hbm, xlo_hbm, xhi_hbm, out_hbm,
              ilo_v, ihi_v, buf0, buf1, sem0, sem1):
    c = lax.axis_index("c")
    s = lax.axis_index("s")
    w = c * NS + s
    pltpu.sync_copy(xlo_hbm, ilo_v)
    pltpu.sync_copy(xhi_hbm, ihi_v)

    nrows = jnp.where(w < XTR, CHKD + 1, CHKD)
    start = jnp.where(w < XTR, w * (CHKD + 1),
                      XTR * (CHKD + 1) + (w - XTR) * CHKD)

    def body(j, _):
        cp0 = pltpu.async_copy(p_hbm.at[ilo_v.at[start + j]], buf0, sem0)
        cp1 = pltpu.async_copy(p_hbm.at[ihi_v.at[start + j]], buf1, sem1)
        cp0.wait()
        cp1.wait()

        def add_row(r, _):
            for k in range(D // 16):
                buf0[r, pl.ds(k * 16, 16)] = (
                    buf0[r, pl.ds(k * 16, 16)] + buf1[r, pl.ds(k * 16, 16)])
            return 0

        lax.fori_loop(0, 128, add_row, 0)
        base = (start + j) * 128
        pltpu.sync_copy(buf0, out_hbm.at[pl.ds(base, 128)])
        return 0

    lax.fori_loop(0, nrows, body, 0)


_gat = functools.partial(
    pl.kernel,
    out_type=jax.ShapeDtypeStruct((XF, D), jnp.float32),
    mesh=_mesh,
    scratch_types=[
        pltpu.VMEM((XR, 128), jnp.int32),
        pltpu.VMEM((XR, 128), jnp.int32),
        pltpu.VMEM((128, D), jnp.float32),
        pltpu.VMEM((128, D), jnp.float32),
        pltpu.SemaphoreType.DMA,
        pltpu.SemaphoreType.DMA,
    ],
)(_gat_body)


_PAD_EDGE = (10000 + (np.arange(EP - E) % (NP - N))).astype(np.int32)


def kernel(features, matrix, x, W, b):
    feat_p = jnp.zeros((NP, D), jnp.float32).at[:N].set(features)
    pad = jnp.asarray(_PAD_EDGE)
    src = jnp.concatenate([matrix[0], pad]).reshape(ER, 128)
    dst = jnp.concatenate([matrix[1], pad]).reshape(ER, 128)

    dh = _deg(dst)
    h2z, dis2 = _tc(feat_p, W, dh.reshape(NC, NP, 1))
    dis = dis2.reshape(NP)
    p = _agg(src, dst, h2z, dis, b)
    xlo = x.reshape(XR, 128)
    xhi = xlo + NP
    out = _gat(p, xlo, xhi)
    return out.reshape(BATCH, FIELDS, D)


# speedup vs baseline: 26.2199x; 1.0469x over previous
import functools

import jax
import jax.numpy as jnp
import numpy as np
from jax import lax
from jax.experimental import pallas as pl
from jax.experimental.pallas import tpu as pltpu
from jax.experimental.pallas import tpu_sc as plsc

N = 10000
D = 128
E = 320000
BATCH = 1024
FIELDS = 26

NC, NS = 2, 16
NP = 10240
EP = 327680
ER = EP // 128
ROWS_T = NP // NS

NCHK = EP // (NC * NS * 128)
HCHK = NCHK // 2
XF = BATCH * FIELDS
XR = XF // 128
CHKD = XR // (NC * NS)
XTR = XR - NC * NS * CHKD

_mesh = plsc.VectorSubcoreMesh(core_axis_name="c", subcore_axis_name="s")


def _deg_body(dst_hbm, out_hbm, idx_v, ones_v, zeros_v, accum_sh):
    c = lax.axis_index("c")
    s = lax.axis_index("s")
    w = c * NS + s

    def fz(i, _):
        zeros_v[pl.ds(i * 16, 16)] = jnp.zeros((16,), jnp.float32)
        return 0

    lax.fori_loop(0, ROWS_T // 16, fz, 0)

    def fo(i, _):
        ones_v[pl.ds(i * 16, 16)] = jnp.ones((16,), jnp.float32)
        return 0

    lax.fori_loop(0, 128 // 16, fo, 0)

    pltpu.sync_copy(zeros_v, accum_sh.at[pl.ds(s * ROWS_T, ROWS_T)])
    pltpu.sync_copy(dst_hbm.at[pl.ds(w * NCHK, NCHK)], idx_v)
    plsc.subcore_barrier()

    def body(j, _):
        pltpu.sync_copy(ones_v, accum_sh.at[idx_v.at[j]], add=True)
        return 0

    lax.fori_loop(0, NCHK, body, 0)
    plsc.subcore_barrier()
    pltpu.sync_copy(accum_sh.at[pl.ds(s * ROWS_T, ROWS_T)],
                    out_hbm.at[pl.ds(c * NP + s * ROWS_T, ROWS_T)])


_deg = functools.partial(
    pl.kernel,
    out_type=jax.ShapeDtypeStruct((NC * NP,), jnp.float32),
    mesh=_mesh,
    scratch_types=[
        pltpu.VMEM((NCHK, 128), jnp.int32),
        pltpu.VMEM((128,), jnp.float32),
        pltpu.VMEM((ROWS_T,), jnp.float32),
        pltpu.VMEM_SHARED((NP,), jnp.float32),
    ],
)(_deg_body)


RB = 1024
GB = NP // RB


def _tc_body(f_ref, w_ref, dh_ref, h2z_ref, dis_ref):
    i = pl.program_id(0)
    deg = 1.0 + dh_ref[0] + dh_ref[1]
    dis = lax.rsqrt(deg)
    dis_ref[...] = dis

    @pl.when(i < GB)
    def _():
        h = jnp.dot(f_ref[...], w_ref[...],
                    preferred_element_type=jnp.float32)
        h2z_ref[...] = h * dis

    @pl.when(i >= GB)
    def _():
        h2z_ref[...] = jnp.zeros((RB, D), jnp.float32)


def _tc(feat_p, W, dh3):
    return pl.pallas_call(
        _tc_body,
        grid=(2 * GB,),
        in_specs=[
            pl.BlockSpec((RB, D), lambda i: (i % GB, 0)),
            pl.BlockSpec((D, D), lambda i: (0, 0)),
            pl.BlockSpec((2, RB, 1), lambda i: (0, i % GB, 0)),
        ],
        out_specs=[
            pl.BlockSpec((RB, D), lambda i: (i, 0)),
            pl.BlockSpec((RB, 1), lambda i: (i % GB, 0)),
        ],
        out_shape=[
            jax.ShapeDtypeStruct((NC * NP, D), jnp.float32),
            jax.ShapeDtypeStruct((NP, 1), jnp.float32),
        ],
    )(feat_p, W, dh3)


def _agg_body(src_hbm, dst_hbm, h2z_hbm, dis_hbm, b_hbm,
              p_hbm,
              sidx, didx, buf_a, buf_b, dis_v, b_v, accum_sh,
              sem_a, sem_b):
    c = lax.axis_index("c")
    s = lax.axis_index("s")
    w = c * NS + s
    r0 = s * ROWS_T

    pltpu.sync_copy(h2z_hbm.at[pl.ds(c * NP + r0, ROWS_T)],
                    accum_sh.at[pl.ds(r0, ROWS_T)])
    plsc.subcore_barrier()

    def half(hf, _):
        base = w * NCHK + hf * HCHK
        pltpu.sync_copy(src_hbm.at[pl.ds(base, HCHK)], sidx)
        pltpu.sync_copy(dst_hbm.at[pl.ds(base, HCHK)], didx)
        pltpu.async_copy(h2z_hbm.at[sidx.at[0]], buf_a, sem_a)

        def body(jj, _):
            j0 = 2 * jj
            pltpu.async_copy(h2z_hbm.at[sidx.at[j0 + 1]], buf_b, sem_b)
            pltpu.make_async_copy(h2z_hbm.at[sidx.at[j0]],
                                  buf_a, sem_a).wait()
            pltpu.sync_copy(buf_a, accum_sh.at[didx.at[j0]], add=True)
            pltpu.async_copy(h2z_hbm.at[sidx.at[j0 + 2]], buf_a, sem_a)
            pltpu.make_async_copy(h2z_hbm.at[sidx.at[j0 + 1]],
                                  buf_b, sem_b).wait()
            pltpu.sync_copy(buf_b, accum_sh.at[didx.at[j0 + 1]], add=True)
            return 0

        lax.fori_loop(0, HCHK // 2 - 1, body, 0)
        pltpu.async_copy(h2z_hbm.at[sidx.at[HCHK - 1]], buf_b, sem_b)
        pltpu.make_async_copy(h2z_hbm.at[sidx.at[HCHK - 2]],
                              buf_a, sem_a).wait()
        pltpu.sync_copy(buf_a, accum_sh.at[didx.at[HCHK - 2]], add=True)
        pltpu.make_async_copy(h2z_hbm.at[sidx.at[HCHK - 1]],
                              buf_b, sem_b).wait()
        pltpu.sync_copy(buf_b, accum_sh.at[didx.at[HCHK - 1]], add=True)
        return 0

    lax.fori_loop(0, 2, half, 0)
    plsc.subcore_barrier()

    pltpu.sync_copy(b_hbm, b_v)
    bmask = jnp.where(c == 0, 1.0, 0.0).astype(jnp.float32)
    bk = [b_v[pl.ds(k * 16, 16)] * bmask for k in range(D // 16)]

    def fin_chunk(t, _):
        rbase = r0 + t * 128
        pltpu.sync_copy(accum_sh.at[pl.ds(rbase, 128)], buf_a)
        pltpu.sync_copy(dis_hbm.at[pl.ds(rbase, 128)], dis_v)

        def fin(g, _):
            dvec = dis_v[pl.ds(g * 16, 16)]
            for rl in range(16):
                r = g * 16 + rl
                dr = dvec[rl]
                for k in range(D // 16):
                    buf_a[r, pl.ds(k * 16, 16)] = (
                        buf_a[r, pl.ds(k * 16, 16)] * dr + bk[k])
            return 0

        lax.fori_loop(0, 128 // 16, fin, 0)
        pltpu.sync_copy(buf_a, p_hbm.at[pl.ds(c * NP + rbase, 128)])
        return 0

    lax.fori_loop(0, ROWS_T // 128, fin_chunk, 0)


_agg = functools.partial(
    pl.kernel,
    out_type=jax.ShapeDtypeStruct((NC * NP, D), jnp.float32),
    mesh=_mesh,
    scratch_types=[
        pltpu.VMEM((HCHK, 128), jnp.int32),
        pltpu.VMEM((HCHK, 128), jnp.int32),
        pltpu.VMEM((128, D), jnp.float32),
        pltpu.VMEM((128, D), jnp.float32),
        pltpu.VMEM((128,), jnp.float32),
        pltpu.VMEM((D,), jnp.float32),
        pltpu.VMEM_SHARED((NP, D), jnp.float32),
        pltpu.SemaphoreType.DMA,
        pltpu.SemaphoreType.DMA,
    ],
)(_agg_body)


def _gat_body(p_hbm, xlo_hbm, xhi_hbm, out_hbm,
              ilo_v, ihi_v, buf0, buf1, buf2, sem0, sem1):
    c = lax.axis_index("c")
    s = lax.axis_index("s")
    w = c * NS + s
    pltpu.sync_copy(xlo_hbm, ilo_v)
    pltpu.sync_copy(xhi_hbm, ihi_v)

    nrows = jnp.where(w < XTR, CHKD + 1, CHKD)
    start = jnp.where(w < XTR, w * (CHKD + 1),
                      XTR * (CHKD + 1) + (w - XTR) * CHKD)

    pltpu.async_copy(p_hbm.at[ilo_v.at[start]], buf0, sem0)
    pltpu.async_copy(p_hbm.at[ihi_v.at[start]], buf1, sem1)

    def body(j, _):
        jn = start + jnp.minimum(j + 1, nrows - 1)
        pltpu.make_async_copy(p_hbm.at[ilo_v.at[start + j]],
                              buf0, sem0).wait()
        pltpu.make_async_copy(p_hbm.at[ihi_v.at[start + j]],
                              buf1, sem1).wait()

        def add_row(r, _):
            for k in range(D // 16):
                buf2[r, pl.ds(k * 16, 16)] = (
                    buf0[r, pl.ds(k * 16, 16)] + buf1[r, pl.ds(k * 16, 16)])
            return 0

        lax.fori_loop(0, 128, add_row, 0)
        pltpu.async_copy(p_hbm.at[ilo_v.at[jn]], buf0, sem0)
        pltpu.async_copy(p_hbm.at[ihi_v.at[jn]], buf1, sem1)
        base = (start + j) * 128
        pltpu.sync_copy(buf2, out_hbm.at[pl.ds(base, 128)])
        return 0

    lax.fori_loop(0, nrows, body, 0)
    pltpu.make_async_copy(p_hbm.at[ilo_v.at[start]], buf0, sem0).wait()
    pltpu.make_async_copy(p_hbm.at[ihi_v.at[start]], buf1, sem1).wait()


_gat = functools.partial(
    pl.kernel,
    out_type=jax.ShapeDtypeStruct((XF, D), jnp.float32),
    mesh=_mesh,
    scratch_types=[
        pltpu.VMEM((XR, 128), jnp.int32),
        pltpu.VMEM((XR, 128), jnp.int32),
        pltpu.VMEM((128, D), jnp.float32),
        pltpu.VMEM((128, D), jnp.float32),
        pltpu.VMEM((128, D), jnp.float32),
        pltpu.SemaphoreType.DMA,
        pltpu.SemaphoreType.DMA,
    ],
)(_gat_body)


_PAD_EDGE = (10000 + (np.arange(EP - E) % (NP - N))).astype(np.int32)


def kernel(features, matrix, x, W, b):
    feat_p = jnp.zeros((NP, D), jnp.float32).at[:N].set(features)
    pad = jnp.asarray(_PAD_EDGE)
    src = jnp.concatenate([matrix[0], pad]).reshape(ER, 128)
    dst = jnp.concatenate([matrix[1], pad]).reshape(ER, 128)

    dh = _deg(dst)
    h2z, dis2 = _tc(feat_p, W, dh.reshape(NC, NP, 1))
    dis = dis2.reshape(NP)
    p = _agg(src, dst, h2z, dis, b)
    xlo = x.reshape(XR, 128)
    xhi = xlo + NP
    out = _gat(p, xlo, xhi)
    return out.reshape(BATCH, FIELDS, D)
